# baseline (device time: 843702 ns/iter reference)
import jax
import jax.numpy as jnp
from jax import lax
from jax.experimental import pallas as pl
from jax.experimental.pallas import tpu as pltpu

N_DEV = 16
NSLOT = 3


def kernel(x, w_mat):
    m_per, k = x.shape
    n = w_mat.shape[1]

    def body(x_ref, w_ref, out_ref, comm_ref, labuf_ref, gbuf_ref,
             send_sems, recv_sems, asend_sems, arecv_sems):
        me = lax.axis_index("i")
        left = lax.rem(me - 1 + N_DEV, N_DEV)
        right = lax.rem(me + 1, N_DEV)

        barrier_sem = pltpu.get_barrier_semaphore()
        for nbr in (left, right):
            pl.semaphore_signal(
                barrier_sem, inc=1,
                device_id=(nbr,), device_id_type=pl.DeviceIdType.MESH,
            )
        pl.semaphore_wait(barrier_sem, 2)

        def matmul(a):
            return lax.dot_general(
                a, w_ref[...], (((1,), (0,)), ((), ())),
                precision=lax.Precision.HIGHEST,
                preferred_element_type=jnp.float32,
            )

        y = matmul(x_ref[...])
        out_ref[pl.ds(me * m_per, m_per), :] = y
        la = jnp.max(jnp.abs(y))

        for h in range(N_DEV - 1):
            src = x_ref if h == 0 else comm_ref.at[(h - 1) % NSLOT]
            rdma = pltpu.make_async_remote_copy(
                src_ref=src,
                dst_ref=comm_ref.at[h % NSLOT],
                send_sem=send_sems.at[h],
                recv_sem=recv_sems.at[h],
                device_id=(right,),
                device_id_type=pl.DeviceIdType.MESH,
            )
            rdma.start()
            rdma.wait()
            origin = lax.rem(me - h - 1 + N_DEV, N_DEV)
            y = matmul(comm_ref[h % NSLOT])
            out_ref[pl.ds(origin * m_per, m_per), :] = y
            la = jnp.maximum(la, jnp.max(jnp.abs(y)))

        labuf_ref[...] = jnp.full((8, 128), la, jnp.float32)
        sends = []
        for d in range(1, N_DEV):
            tgt = lax.rem(me + d, N_DEV)
            rdma = pltpu.make_async_remote_copy(
                src_ref=labuf_ref,
                dst_ref=gbuf_ref.at[d - 1],
                send_sem=asend_sems.at[d - 1],
                recv_sem=arecv_sems.at[d - 1],
                device_id=(tgt,),
                device_id_type=pl.DeviceIdType.MESH,
            )
            rdma.start()
            sends.append(rdma)
        for rdma in sends:
            rdma.wait_send()
        for rdma in sends:
            rdma.wait_recv()

        gmax = jnp.maximum(la, jnp.max(gbuf_ref[...]))
        scale = gmax / 127.0
        yq = jnp.clip(jnp.round(out_ref[...] / scale), -127.0, 127.0)
        out_ref[...] = yq * scale

    return pl.pallas_call(
        body,
        out_shape=jax.ShapeDtypeStruct((N_DEV * m_per, n), jnp.float32),
        in_specs=[
            pl.BlockSpec(memory_space=pltpu.VMEM),
            pl.BlockSpec(memory_space=pltpu.VMEM),
        ],
        out_specs=pl.BlockSpec(memory_space=pltpu.VMEM),
        scratch_shapes=[
            pltpu.VMEM((NSLOT, m_per, k), jnp.float32),
            pltpu.VMEM((8, 128), jnp.float32),
            pltpu.VMEM((N_DEV - 1, 8, 128), jnp.float32),
            pltpu.SemaphoreType.DMA((N_DEV - 1,)),
            pltpu.SemaphoreType.DMA((N_DEV - 1,)),
            pltpu.SemaphoreType.DMA((N_DEV - 1,)),
            pltpu.SemaphoreType.DMA((N_DEV - 1,)),
        ],
        compiler_params=pltpu.CompilerParams(collective_id=0),
    )(x, w_mat)


# device time: 381033 ns/iter; 2.2142x vs baseline; 2.2142x over previous
import jax
import jax.numpy as jnp
from jax import lax
from jax.experimental import pallas as pl
from jax.experimental.pallas import tpu as pltpu

N_DEV = 16
NSLOT = 3
NHOP = 8
HALF = 128


def kernel(x, w_mat):
    m_per, k = x.shape
    n = w_mat.shape[1]

    def body(x_ref, w_ref, out_ref, comm_f, comm_b, labuf_ref, gbuf_ref,
             fsend, frecv, bsend, brecv, asend, arecv):
        me = lax.axis_index("i")
        left = lax.rem(me - 1 + N_DEV, N_DEV)
        right = lax.rem(me + 1, N_DEV)

        barrier_sem = pltpu.get_barrier_semaphore()
        for nbr in (left, right):
            pl.semaphore_signal(
                barrier_sem, inc=1,
                device_id=(nbr,), device_id_type=pl.DeviceIdType.MESH,
            )
        pl.semaphore_wait(barrier_sem, 2)

        def matmul(a):
            return lax.dot_general(
                a, w_ref[...], (((1,), (0,)), ((), ())),
                precision=lax.Precision.HIGHEST,
                preferred_element_type=jnp.float32,
            )

        def mk_fwd(t):
            if t == 0:
                src = x_ref
            elif t < NHOP - 1:
                src = comm_f.at[(t - 1) % NSLOT]
            else:
                src = comm_f.at[(t - 1) % NSLOT, pl.ds(0, HALF)]
            if t < NHOP - 1:
                dst = comm_f.at[t % NSLOT]
            else:
                dst = comm_f.at[t % NSLOT, pl.ds(0, HALF)]
            return pltpu.make_async_remote_copy(
                src_ref=src, dst_ref=dst,
                send_sem=fsend.at[t], recv_sem=frecv.at[t],
                device_id=(right,), device_id_type=pl.DeviceIdType.MESH,
            )

        def mk_bwd(t):
            if t == 0:
                src = x_ref
            elif t < NHOP - 1:
                src = comm_b.at[(t - 1) % NSLOT]
            else:
                src = comm_b.at[(t - 1) % NSLOT, pl.ds(HALF, HALF)]
            if t < NHOP - 1:
                dst = comm_b.at[t % NSLOT]
            else:
                dst = comm_b.at[t % NSLOT, pl.ds(HALF, HALF)]
            return pltpu.make_async_remote_copy(
                src_ref=src, dst_ref=dst,
                send_sem=bsend.at[t], recv_sem=brecv.at[t],
                device_id=(left,), device_id_type=pl.DeviceIdType.MESH,
            )

        f = [None] * NHOP
        b = [None] * NHOP
        f[0] = mk_fwd(0)
        f[0].start()
        b[0] = mk_bwd(0)
        b[0].start()

        y = matmul(x_ref[...])
        out_ref[pl.ds(me * m_per, m_per), :] = y
        la = jnp.max(jnp.abs(y))

        for t in range(NHOP):
            f[t].wait_recv()
            if t < NHOP - 1:
                f[t + 1] = mk_fwd(t + 1)
                f[t + 1].start()
            b[t].wait_recv()
            if t < NHOP - 1:
                b[t + 1] = mk_bwd(t + 1)
                b[t + 1].start()

            o_f = lax.rem(me - t - 1 + N_DEV, N_DEV)
            if t < NHOP - 1:
                y = matmul(comm_f[t % NSLOT])
                out_ref[pl.ds(o_f * m_per, m_per), :] = y
            else:
                y = matmul(comm_f[t % NSLOT, pl.ds(0, HALF)])
                out_ref[pl.ds(o_f * m_per, HALF), :] = y
            la = jnp.maximum(la, jnp.max(jnp.abs(y)))

            o_b = lax.rem(me + t + 1, N_DEV)
            if t < NHOP - 1:
                y = matmul(comm_b[t % NSLOT])
                out_ref[pl.ds(o_b * m_per, m_per), :] = y
            else:
                y = matmul(comm_b[t % NSLOT, pl.ds(HALF, HALF)])
                out_ref[pl.ds(o_b * m_per + HALF, HALF), :] = y
            la = jnp.maximum(la, jnp.max(jnp.abs(y)))

        for t in range(NHOP):
            f[t].wait_send()
            b[t].wait_send()

        labuf_ref[...] = jnp.full((8, 128), la, jnp.float32)
        sends = []
        for d in range(1, N_DEV):
            tgt = lax.rem(me + d, N_DEV)
            rdma = pltpu.make_async_remote_copy(
                src_ref=labuf_ref,
                dst_ref=gbuf_ref.at[d - 1],
                send_sem=asend.at[d - 1],
                recv_sem=arecv.at[d - 1],
                device_id=(tgt,),
                device_id_type=pl.DeviceIdType.MESH,
            )
            rdma.start()
            sends.append(rdma)
        for rdma in sends:
            rdma.wait_send()
        for rdma in sends:
            rdma.wait_recv()

        gmax = jnp.maximum(la, jnp.max(gbuf_ref[...]))
        scale = gmax / 127.0
        yq = jnp.clip(jnp.round(out_ref[...] / scale), -127.0, 127.0)
        out_ref[...] = yq * scale

    return pl.pallas_call(
        body,
        out_shape=jax.ShapeDtypeStruct((N_DEV * m_per, n), jnp.float32),
        in_specs=[
            pl.BlockSpec(memory_space=pltpu.VMEM),
            pl.BlockSpec(memory_space=pltpu.VMEM),
        ],
        out_specs=pl.BlockSpec(memory_space=pltpu.VMEM),
        scratch_shapes=[
            pltpu.VMEM((NSLOT, m_per, k), jnp.float32),
            pltpu.VMEM((NSLOT, m_per, k), jnp.float32),
            pltpu.VMEM((8, 128), jnp.float32),
            pltpu.VMEM((N_DEV - 1, 8, 128), jnp.float32),
            pltpu.SemaphoreType.DMA((NHOP,)),
            pltpu.SemaphoreType.DMA((NHOP,)),
            pltpu.SemaphoreType.DMA((NHOP,)),
            pltpu.SemaphoreType.DMA((NHOP,)),
            pltpu.SemaphoreType.DMA((N_DEV - 1,)),
            pltpu.SemaphoreType.DMA((N_DEV - 1,)),
        ],
        compiler_params=pltpu.CompilerParams(collective_id=0),
    )(x, w_mat)


# device time: 370961 ns/iter; 2.2744x vs baseline; 1.0272x over previous
import jax
import jax.numpy as jnp
from jax import lax
from jax.experimental import pallas as pl
from jax.experimental.pallas import tpu as pltpu

N_DEV = 16
NSLOT = 3
NHOP = 8
HALF = 128


def kernel(x, w_mat):
    m_per, k = x.shape
    n = w_mat.shape[1]

    def body(x_ref, w_ref, out_ref, comm_f, comm_b, labuf_ref, gbuf_ref,
             fsend, frecv, bsend, brecv, asend, arecv):
        me = lax.axis_index("i")
        left = lax.rem(me - 1 + N_DEV, N_DEV)
        right = lax.rem(me + 1, N_DEV)

        barrier_sem = pltpu.get_barrier_semaphore()
        for nbr in (left, right):
            pl.semaphore_signal(
                barrier_sem, inc=1,
                device_id=(nbr,), device_id_type=pl.DeviceIdType.MESH,
            )
        pl.semaphore_wait(barrier_sem, 2)

        def matmul(a):
            return lax.dot_general(
                a, w_ref[...], (((1,), (0,)), ((), ())),
                precision=lax.Precision.HIGHEST,
                preferred_element_type=jnp.float32,
            )

        def rows(t, fwd):
            if t < NHOP - 1:
                return 0, m_per
            return (0, HALF) if fwd else (HALF, HALF)

        def mk(t, h, fwd):
            comm = comm_f if fwd else comm_b
            base, ln = rows(t, fwd)
            idx = h if fwd else 1 - h
            start, mlen = base + idx * (ln // 2), ln // 2
            if t == 0:
                src = x_ref.at[pl.ds(start, mlen)]
            else:
                src = comm.at[(t - 1) % NSLOT, pl.ds(start, mlen)]
            dst = comm.at[t % NSLOT, pl.ds(start, mlen)]
            ssem = fsend if fwd else bsend
            rsem = frecv if fwd else brecv
            return pltpu.make_async_remote_copy(
                src_ref=src, dst_ref=dst,
                send_sem=ssem.at[t, h], recv_sem=rsem.at[t, h],
                device_id=(right if fwd else left,),
                device_id_type=pl.DeviceIdType.MESH,
            )

        f = [[None, None] for _ in range(NHOP)]
        b = [[None, None] for _ in range(NHOP)]
        for h in (0, 1):
            f[0][h] = mk(0, h, True)
            f[0][h].start()
            b[0][h] = mk(0, h, False)
            b[0][h].start()

        y = matmul(x_ref[...])
        out_ref[pl.ds(me * m_per, m_per), :] = y
        la = jnp.max(jnp.abs(y))

        for t in range(NHOP):
            for h in (0, 1):
                f[t][h].wait_recv()
                if t < NHOP - 1:
                    f[t + 1][h] = mk(t + 1, h, True)
                    f[t + 1][h].start()
            for h in (0, 1):
                b[t][h].wait_recv()
                if t < NHOP - 1:
                    b[t + 1][h] = mk(t + 1, h, False)
                    b[t + 1][h].start()

            o_f = lax.rem(me - t - 1 + N_DEV, N_DEV)
            base, ln = rows(t, True)
            y = matmul(comm_f[t % NSLOT, pl.ds(base, ln)])
            out_ref[pl.ds(o_f * m_per + base, ln), :] = y
            la = jnp.maximum(la, jnp.max(jnp.abs(y)))

            o_b = lax.rem(me + t + 1, N_DEV)
            base, ln = rows(t, False)
            y = matmul(comm_b[t % NSLOT, pl.ds(base, ln)])
            out_ref[pl.ds(o_b * m_per + base, ln), :] = y
            la = jnp.maximum(la, jnp.max(jnp.abs(y)))

        for t in range(NHOP):
            for h in (0, 1):
                f[t][h].wait_send()
                b[t][h].wait_send()

        labuf_ref[...] = jnp.full((8, 128), la, jnp.float32)
        sends = []
        for d in range(1, N_DEV):
            tgt = lax.rem(me + d, N_DEV)
            rdma = pltpu.make_async_remote_copy(
                src_ref=labuf_ref,
                dst_ref=gbuf_ref.at[d - 1],
                send_sem=asend.at[d - 1],
                recv_sem=arecv.at[d - 1],
                device_id=(tgt,),
                device_id_type=pl.DeviceIdType.MESH,
            )
            rdma.start()
            sends.append(rdma)
        for rdma in sends:
            rdma.wait_send()
        for rdma in sends:
            rdma.wait_recv()

        gmax = jnp.maximum(la, jnp.max(gbuf_ref[...]))
        inv = 127.0 / gmax
        yq = jnp.clip(jnp.round(out_ref[...] * inv), -127.0, 127.0)
        out_ref[...] = yq * (gmax / 127.0)

    return pl.pallas_call(
        body,
        out_shape=jax.ShapeDtypeStruct((N_DEV * m_per, n), jnp.float32),
        in_specs=[
            pl.BlockSpec(memory_space=pltpu.VMEM),
            pl.BlockSpec(memory_space=pltpu.VMEM),
        ],
        out_specs=pl.BlockSpec(memory_space=pltpu.VMEM),
        scratch_shapes=[
            pltpu.VMEM((NSLOT, m_per, k), jnp.float32),
            pltpu.VMEM((NSLOT, m_per, k), jnp.float32),
            pltpu.VMEM((8, 128), jnp.float32),
            pltpu.VMEM((N_DEV - 1, 8, 128), jnp.float32),
            pltpu.SemaphoreType.DMA((NHOP, 2)),
            pltpu.SemaphoreType.DMA((NHOP, 2)),
            pltpu.SemaphoreType.DMA((NHOP, 2)),
            pltpu.SemaphoreType.DMA((NHOP, 2)),
            pltpu.SemaphoreType.DMA((N_DEV - 1,)),
            pltpu.SemaphoreType.DMA((N_DEV - 1,)),
        ],
        compiler_params=pltpu.CompilerParams(collective_id=0),
    )(x, w_mat)
